# baseline (speedup 1.0000x reference)
import functools
import math

import jax
import jax.numpy as jnp
from jax.experimental import pallas as pl

_SIZE = 100000
_PAD = 0
_SMOOTHING = 0.1
_CONF = 1.0 - _SMOOTHING
_EPS = _SMOOTHING / (_SIZE - 2)
_C1 = _SMOOTHING * math.log(_EPS) + _CONF * math.log(_CONF)

_VB = 2000
_K = 2


def _loss_kernel(*refs):
    xt_refs = refs[:_K]
    tgt_ref, out_ref = refs[_K], refs[_K + 1]
    i = pl.program_id(0)

    @pl.when(i == 0)
    def _init():
        out_ref[...] = jnp.zeros((1, 1), jnp.float32)

    tgt = tgt_ref[...]
    valid = tgt != _PAD
    acc = jnp.zeros((1, 1), jnp.float32)
    for k in range(_K):
        xb = xt_refs[k][...]
        jrow = jax.lax.broadcasted_iota(jnp.int32, xb.shape, 0)
        tloc = tgt - (i * _K + k) * _VB
        w = jnp.where(jrow == tloc, -_CONF, -_EPS)
        s_cols = jnp.sum(w * xb, axis=0, keepdims=True)
        acc += jnp.sum(jnp.where(valid, s_cols, 0.0), axis=1, keepdims=True)

    @pl.when(i == 0)
    def _pad_col_and_const():
        x0 = xt_refs[0][0:1, :]
        extra = jnp.where(valid, _EPS * x0 + _C1, 0.0)
        out_ref[...] += jnp.sum(extra, axis=1, keepdims=True)

    out_ref[...] += acc


@functools.partial(jax.jit, static_argnames=())
def kernel(x, target):
    n = x.shape[0]
    xt = x.T
    tgt = target.astype(jnp.int32).reshape(1, n)
    xt_specs = [
        pl.BlockSpec((_VB, n), functools.partial(lambda k, i: (i * _K + k, 0), k))
        for k in range(_K)
    ]
    out = pl.pallas_call(
        _loss_kernel,
        grid=(_SIZE // (_K * _VB),),
        in_specs=xt_specs + [pl.BlockSpec((1, n), lambda i: (0, 0))],
        out_specs=pl.BlockSpec((1, 1), lambda i: (0, 0)),
        out_shape=jax.ShapeDtypeStruct((1, 1), jnp.float32),
    )(*([xt] * _K), tgt)
    return out[0, 0]


# speedup vs baseline: 1.0095x; 1.0095x over previous
import functools
import math

import jax
import jax.numpy as jnp
from jax.experimental import pallas as pl
from jax.experimental.pallas import tpu as pltpu

_SIZE = 100000
_PAD = 0
_SMOOTHING = 0.1
_CONF = 1.0 - _SMOOTHING
_EPS = _SMOOTHING / (_SIZE - 2)
_C1 = _SMOOTHING * math.log(_EPS) + _CONF * math.log(_CONF)

_VB = 2000
_NBUF = 4
_NSTEP = _SIZE // _VB


def _loss_kernel(xt_hbm, tgt_ref, out_ref, *scratch):
    bufs = scratch[:_NBUF]
    sem = scratch[_NBUF]

    def _start(idx, b):
        pltpu.make_async_copy(
            xt_hbm.at[pl.ds(idx * _VB, _VB), :], bufs[b], sem.at[b]
        ).start()

    def _wait(b):
        pltpu.make_async_copy(xt_hbm.at[pl.ds(0, _VB), :], bufs[b], sem.at[b]).wait()

    for b in range(_NBUF):
        _start(b, b)

    tgt = tgt_ref[...]
    valid = tgt != _PAD
    out_ref[...] = jnp.zeros((1, 1), jnp.float32)

    def _block(idx, b):
        _wait(b)
        xb = bufs[b][...]
        jrow = jax.lax.broadcasted_iota(jnp.int32, xb.shape, 0)
        tloc = tgt - idx * _VB
        w = jnp.where(jrow == tloc, -_CONF, -_EPS)
        s_cols = jnp.sum(w * xb, axis=0, keepdims=True)

        @pl.when(idx == 0)
        def _pad_col_and_const():
            x0 = bufs[b][0:1, :]
            extra = jnp.where(valid, _EPS * x0 + _C1, 0.0)
            out_ref[...] += jnp.sum(extra, axis=1, keepdims=True)

        out_ref[...] += jnp.sum(
            jnp.where(valid, s_cols, 0.0), axis=1, keepdims=True
        )

        @pl.when(idx + _NBUF < _NSTEP)
        def _refill():
            _start(idx + _NBUF, b)

    def _body(g, carry):
        for b in range(_NBUF):
            _block(g * _NBUF + b, b)
        return carry

    jax.lax.fori_loop(0, _NSTEP // _NBUF, _body, 0)
    for r in range(_NSTEP - (_NSTEP // _NBUF) * _NBUF):
        _block((_NSTEP // _NBUF) * _NBUF + r, r)


@functools.partial(jax.jit, static_argnames=())
def kernel(x, target):
    n = x.shape[0]
    xt = x.T
    tgt = target.astype(jnp.int32).reshape(1, n)
    out = pl.pallas_call(
        _loss_kernel,
        in_specs=[
            pl.BlockSpec(memory_space=pltpu.MemorySpace.HBM),
            pl.BlockSpec(memory_space=pltpu.MemorySpace.VMEM),
        ],
        out_specs=pl.BlockSpec(memory_space=pltpu.MemorySpace.VMEM),
        out_shape=jax.ShapeDtypeStruct((1, 1), jnp.float32),
        scratch_shapes=[pltpu.VMEM((_VB, n), jnp.float32) for _ in range(_NBUF)]
        + [pltpu.SemaphoreType.DMA((_NBUF,))],
    )(xt, tgt)
    return out[0, 0]


# speedup vs baseline: 1.0338x; 1.0240x over previous
import functools
import math

import jax
import jax.numpy as jnp
from jax.experimental import pallas as pl

_SIZE = 100000
_PAD = 0
_SMOOTHING = 0.1
_CONF = 1.0 - _SMOOTHING
_EPS = _SMOOTHING / (_SIZE - 2)
_C1 = _SMOOTHING * math.log(_EPS) + _CONF * math.log(_CONF)

_VB = 5000


def _loss_kernel(xt_ref, tgt_ref, out_ref):
    i = pl.program_id(0)

    @pl.when(i == 0)
    def _init():
        out_ref[...] = jnp.zeros((1, 1), jnp.float32)

    xb = xt_ref[...]
    tgt = tgt_ref[...]
    valid = tgt != _PAD
    jrow = jax.lax.broadcasted_iota(jnp.int32, xb.shape, 0)
    tloc = tgt - i * _VB
    hit = jnp.where(jrow == tloc, xb, 0.0)
    ones = jnp.ones((1, _VB), jnp.float32)
    dn = (((1,), (0,)), ((), ()))
    s_all = jax.lax.dot_general(ones, xb, dn,
                                preferred_element_type=jnp.float32)
    s_hit = jax.lax.dot_general(ones, hit, dn,
                                preferred_element_type=jnp.float32)
    s_cols = -_EPS * s_all - (_CONF - _EPS) * s_hit
    acc = jnp.sum(jnp.where(valid, s_cols, 0.0), axis=1, keepdims=True)

    @pl.when(i == 0)
    def _pad_col_and_const():
        x0 = xt_ref[0:1, :]
        extra = jnp.where(valid, _EPS * x0 + _C1, 0.0)
        out_ref[...] += jnp.sum(extra, axis=1, keepdims=True)

    out_ref[...] += acc


@functools.partial(jax.jit, static_argnames=())
def kernel(x, target):
    n = x.shape[0]
    xt = x.T
    tgt = target.astype(jnp.int32).reshape(1, n)
    out = pl.pallas_call(
        _loss_kernel,
        grid=(_SIZE // _VB,),
        in_specs=[
            pl.BlockSpec((_VB, n), lambda i: (i, 0)),
            pl.BlockSpec((1, n), lambda i: (0, 0)),
        ],
        out_specs=pl.BlockSpec((1, 1), lambda i: (0, 0)),
        out_shape=jax.ShapeDtypeStruct((1, 1), jnp.float32),
    )(xt, tgt)
    return out[0, 0]


# speedup vs baseline: 1.0340x; 1.0003x over previous
import functools
import math

import jax
import jax.numpy as jnp
from jax.experimental import pallas as pl

_SIZE = 100000
_PAD = 0
_SMOOTHING = 0.1
_CONF = 1.0 - _SMOOTHING
_EPS = _SMOOTHING / (_SIZE - 2)
_C1 = _SMOOTHING * math.log(_EPS) + _CONF * math.log(_CONF)

_VB = 5000


def _loss_kernel(xt_ref, tgt_ref, out_ref):
    i = pl.program_id(0)

    @pl.when(i == 0)
    def _init():
        out_ref[...] = jnp.zeros((1, 1), jnp.float32)

    xb = xt_ref[...]
    tgt = tgt_ref[...]
    valid = tgt != _PAD
    jrow = jax.lax.broadcasted_iota(jnp.int32, xb.shape, 0)
    tloc = tgt - i * _VB
    hit = jnp.where(jrow == tloc, xb, 0.0)
    ones = jnp.ones((1, _VB), jnp.float32)
    dn = (((1,), (0,)), ((), ()))
    s_all = jax.lax.dot_general(ones, xb, dn,
                                preferred_element_type=jnp.float32)
    s_hit = jnp.sum(hit, axis=0, keepdims=True)
    s_cols = -_EPS * s_all - (_CONF - _EPS) * s_hit
    acc = jnp.sum(jnp.where(valid, s_cols, 0.0), axis=1, keepdims=True)

    @pl.when(i == 0)
    def _pad_col_and_const():
        x0 = xt_ref[0:1, :]
        extra = jnp.where(valid, _EPS * x0 + _C1, 0.0)
        out_ref[...] += jnp.sum(extra, axis=1, keepdims=True)

    out_ref[...] += acc


@functools.partial(jax.jit, static_argnames=())
def kernel(x, target):
    n = x.shape[0]
    xt = x.T
    tgt = target.astype(jnp.int32).reshape(1, n)
    out = pl.pallas_call(
        _loss_kernel,
        grid=(_SIZE // _VB,),
        in_specs=[
            pl.BlockSpec((_VB, n), lambda i: (i, 0)),
            pl.BlockSpec((1, n), lambda i: (0, 0)),
        ],
        out_specs=pl.BlockSpec((1, 1), lambda i: (0, 0)),
        out_shape=jax.ShapeDtypeStruct((1, 1), jnp.float32),
    )(xt, tgt)
    return out[0, 0]
